# baseline (speedup 1.0000x reference)
import functools

import jax
import jax.numpy as jnp
from jax import lax
from jax.experimental import pallas as pl
from jax.experimental.pallas import tpu as pltpu
from jax.experimental.pallas import tpu_sc as plsc

V = 100000
D = 300
SEQ = 50
B = 4096

MAIN = 256
TAIL = D - MAIN
TP = 128

NC = 2
NS = 16
NW = NC * NS
BPW = B // NW
CHUNK = 64
NCHUNK = BPW // CHUNK
NJ = SEQ // 2
GR = 2 * CHUNK

LANES = 16
UNROLL = 4

_sc_mesh = plsc.VectorSubcoreMesh(
    core_axis_name="c", subcore_axis_name="s", num_cores=NC, num_subcores=NS
)


@functools.partial(
    pl.kernel,
    out_type=(
        jax.ShapeDtypeStruct((B, MAIN), jnp.float32),
        jax.ShapeDtypeStruct((B, MAIN), jnp.float32),
        jax.ShapeDtypeStruct((B, TP), jnp.float32),
    ),
    mesh=_sc_mesh,
    scratch_types=[
        pltpu.VMEM((NJ, GR), jnp.int32),
        [pltpu.VMEM((GR, MAIN), jnp.float32) for _ in range(2)],
        pltpu.VMEM((GR, TP), jnp.float32),
        pltpu.VMEM((CHUNK, MAIN), jnp.float32),
        pltpu.VMEM((CHUNK, TP), jnp.float32),
        [pltpu.SemaphoreType.DMA for _ in range(3)],
    ],
)
def _sc_pool(inpr_hbm, lut_hbm, slut_hbm, tail_hbm,
             e1_hbm, e2_hbm, et_hbm,
             idx_c, bufM, bufT, accM, accT, sems):
    wid = lax.axis_index("s") * NC + lax.axis_index("c")
    zero = jnp.zeros((LANES,), jnp.float32)
    tsem = sems[2]

    def issue_main(tbl, j, slot):
        pltpu.async_copy(tbl.at[:, pl.ds(0, MAIN)].at[idx_c.at[j]],
                         bufM[slot], sems[slot])

    def wait_main(slot):
        pltpu.make_async_copy(
            lut_hbm.at[:, pl.ds(0, MAIN)].at[idx_c.at[0]],
            bufM[slot], sems[slot]).wait()

    def issue_tail(j):
        pltpu.async_copy(tail_hbm.at[idx_c.at[j]], bufT, tsem)

    def wait_tail():
        pltpu.make_async_copy(tail_hbm.at[idx_c.at[0]], bufT, tsem).wait()

    def accum_main(slot):
        def r_body(r, inner):
            for k in range(MAIN // LANES):
                o = pl.ds(k * LANES, LANES)
                x = bufM[slot][r, o] + bufM[slot][CHUNK + r, o]
                plsc.addupdate(accM.at[r, o], x)
            return inner
        lax.fori_loop(0, CHUNK, r_body, 0, unroll=UNROLL)

    def accum_tail():
        def r_body(r, inner):
            for k in range(TP // LANES):
                o = pl.ds(k * LANES, LANES)
                x = bufT[r, o] + bufT[CHUNK + r, o]
                plsc.addupdate(accT.at[r, o], x)
            return inner
        lax.fori_loop(0, CHUNK, r_body, 0, unroll=UNROLL)

    def run_pass(tbl, with_tail):
        issue_main(tbl, 0, 0)
        if with_tail:
            issue_tail(0)

        def zero_body(r, carry):
            for k in range(MAIN // LANES):
                accM[r, pl.ds(k * LANES, LANES)] = zero
            if with_tail:
                for k in range(TP // LANES):
                    accT[r, pl.ds(k * LANES, LANES)] = zero
            return carry

        lax.fori_loop(0, CHUNK, zero_body, 0, unroll=UNROLL)

        def j_body(j, carry):
            for par in range(2):
                jj = 2 * j + par

                @pl.when(jj + 1 < NJ)
                def _():
                    issue_main(tbl, jj + 1, (par + 1) % 2)

                wait_main(par)
                accum_main(par)
                if with_tail:
                    wait_tail()
                    accum_tail()

                    @pl.when(jj + 1 < NJ)
                    def _():
                        issue_tail(jj + 1)
            return carry

        lax.fori_loop(0, NJ // 2, j_body, 0)
        wait_main(0)
        accum_main(0)
        if with_tail:
            wait_tail()
            accum_tail()

    for c in range(NCHUNK):
        base = wid * BPW + c * CHUNK
        pltpu.sync_copy(inpr_hbm.at[wid, c], idx_c)
        run_pass(lut_hbm, True)
        pltpu.sync_copy(accM, e1_hbm.at[pl.ds(base, CHUNK), :])
        pltpu.sync_copy(accT, et_hbm.at[pl.ds(base, CHUNK), :])
        run_pass(slut_hbm, False)
        pltpu.sync_copy(accM, e2_hbm.at[pl.ds(base, CHUNK), :])


MB = 512


def _mlp_body(e1_ref, e2_ref, et_ref, w1a_ref, w1b_ref, w1t_ref,
              b1_ref, w2_ref, b2_ref, out_ref):
    h = jnp.dot(e1_ref[...], w1a_ref[...], preferred_element_type=jnp.float32)
    h = h + jnp.dot(e2_ref[...], w1b_ref[...], preferred_element_type=jnp.float32)
    h = h + jnp.dot(et_ref[...], w1t_ref[...], preferred_element_type=jnp.float32)
    h = jnp.maximum(h + b1_ref[...], 0.0)
    out_ref[...] = jnp.dot(h, w2_ref[...], preferred_element_type=jnp.float32) + b2_ref[...]


_mlp = pl.pallas_call(
    _mlp_body,
    grid=(B // MB,),
    in_specs=[
        pl.BlockSpec((MB, MAIN), lambda i: (i, 0)),
        pl.BlockSpec((MB, MAIN), lambda i: (i, 0)),
        pl.BlockSpec((MB, TP), lambda i: (i, 0)),
        pl.BlockSpec((MAIN, 2 * D), lambda i: (0, 0)),
        pl.BlockSpec((MAIN, 2 * D), lambda i: (0, 0)),
        pl.BlockSpec((TP, 2 * D), lambda i: (0, 0)),
        pl.BlockSpec((1, 2 * D), lambda i: (0, 0)),
        pl.BlockSpec((2 * D, 1), lambda i: (0, 0)),
        pl.BlockSpec((1, 1), lambda i: (0, 0)),
    ],
    out_specs=pl.BlockSpec((MB, 1), lambda i: (i, 0)),
    out_shape=jax.ShapeDtypeStruct((B, 1), jnp.float32),
)


def kernel(input, lut, static_lut, W1, b1, W2, b2):
    tail = jnp.pad(
        jnp.concatenate([lut[:, MAIN:], static_lut[:, MAIN:]], axis=1),
        ((0, 0), (0, TP - 2 * TAIL)))
    inpr = (input.reshape(NJ, 2, NW, NCHUNK, CHUNK)
            .transpose(2, 3, 0, 1, 4)
            .reshape(NW, NCHUNK, NJ, GR))
    e1m, e2m, et = _sc_pool(inpr, lut, static_lut, tail)
    w1t = jnp.pad(
        jnp.concatenate([W1[MAIN:D], W1[D + MAIN:]], axis=0),
        ((0, TP - 2 * TAIL), (0, 0)))
    out = _mlp(e1m, e2m, et, W1[:MAIN], W1[D:D + MAIN], w1t,
               b1.reshape(1, 2 * D), W2, b2.reshape(1, 1))
    return out.reshape(B)


# speedup vs baseline: 1.4114x; 1.4114x over previous
import functools

import jax
import jax.numpy as jnp
from jax import lax
from jax.experimental import pallas as pl
from jax.experimental.pallas import tpu as pltpu
from jax.experimental.pallas import tpu_sc as plsc

V = 100000
D = 300
SEQ = 50
B = 4096

MAIN = 256
TAIL = D - MAIN
TP = 128

NC = 2
NS = 16
NW = NC * NS
BPW = B // NW
CHUNK = 64
NCHUNK = BPW // CHUNK
NJ = SEQ // 2
GR = 2 * CHUNK

LANES = 16
UNROLL = 1

_sc_mesh = plsc.VectorSubcoreMesh(
    core_axis_name="c", subcore_axis_name="s", num_cores=NC, num_subcores=NS
)


@functools.partial(
    pl.kernel,
    out_type=(
        jax.ShapeDtypeStruct((B, MAIN), jnp.float32),
        jax.ShapeDtypeStruct((B, MAIN), jnp.float32),
        jax.ShapeDtypeStruct((B, TP), jnp.float32),
    ),
    mesh=_sc_mesh,
    scratch_types=[
        pltpu.VMEM((NJ, GR), jnp.int32),
        [pltpu.VMEM((GR, MAIN), jnp.float32) for _ in range(2)],
        pltpu.VMEM((GR, TP), jnp.float32),
        pltpu.VMEM((CHUNK, MAIN), jnp.float32),
        pltpu.VMEM((CHUNK, TP), jnp.float32),
        [pltpu.SemaphoreType.DMA for _ in range(3)],
    ],
)
def _sc_pool(inpr_hbm, lut_hbm, slut_hbm, tail_hbm,
             e1_hbm, e2_hbm, et_hbm,
             idx_c, bufM, bufT, accM, accT, sems):
    wid = lax.axis_index("s") * NC + lax.axis_index("c")
    zero = jnp.zeros((LANES,), jnp.float32)
    tsem = sems[2]

    def issue_main(tbl, j, slot):
        pltpu.async_copy(tbl.at[:, pl.ds(0, MAIN)].at[idx_c.at[j]],
                         bufM[slot], sems[slot])

    def wait_main(slot):
        pltpu.make_async_copy(
            lut_hbm.at[:, pl.ds(0, MAIN)].at[idx_c.at[0]],
            bufM[slot], sems[slot]).wait()

    def issue_tail(j):
        pltpu.async_copy(tail_hbm.at[idx_c.at[j]], bufT, tsem)

    def wait_tail():
        pltpu.make_async_copy(tail_hbm.at[idx_c.at[0]], bufT, tsem).wait()

    def accum_main(slot):
        def r_body(r, inner):
            for k in range(MAIN // LANES):
                o = pl.ds(k * LANES, LANES)
                x = bufM[slot][r, o] + bufM[slot][CHUNK + r, o]
                plsc.addupdate(accM.at[r, o], x)
            return inner
        lax.fori_loop(0, CHUNK, r_body, 0, unroll=UNROLL)

    def accum_tail():
        def r_body(r, inner):
            for k in range(TP // LANES):
                o = pl.ds(k * LANES, LANES)
                x = bufT[r, o] + bufT[CHUNK + r, o]
                plsc.addupdate(accT.at[r, o], x)
            return inner
        lax.fori_loop(0, CHUNK, r_body, 0, unroll=UNROLL)

    def run_pass(tbl, with_tail):
        issue_main(tbl, 0, 0)
        if with_tail:
            issue_tail(0)

        def zero_body(r, carry):
            for k in range(MAIN // LANES):
                accM[r, pl.ds(k * LANES, LANES)] = zero
            if with_tail:
                for k in range(TP // LANES):
                    accT[r, pl.ds(k * LANES, LANES)] = zero
            return carry

        lax.fori_loop(0, CHUNK, zero_body, 0, unroll=UNROLL)

        def j_body(j, carry):
            for par in range(2):
                jj = 2 * j + par

                @pl.when(jj + 1 < NJ)
                def _():
                    issue_main(tbl, jj + 1, (par + 1) % 2)

                wait_main(par)
                accum_main(par)
                if with_tail:
                    wait_tail()
                    accum_tail()

                    @pl.when(jj + 1 < NJ)
                    def _():
                        issue_tail(jj + 1)
            return carry

        lax.fori_loop(0, NJ // 2, j_body, 0)
        wait_main(0)
        accum_main(0)
        if with_tail:
            wait_tail()
            accum_tail()

    for c in range(NCHUNK):
        base = wid * BPW + c * CHUNK
        pltpu.sync_copy(inpr_hbm.at[wid, c], idx_c)
        run_pass(lut_hbm, True)
        pltpu.sync_copy(accM, e1_hbm.at[pl.ds(base, CHUNK), :])
        pltpu.sync_copy(accT, et_hbm.at[pl.ds(base, CHUNK), :])
        run_pass(slut_hbm, False)
        pltpu.sync_copy(accM, e2_hbm.at[pl.ds(base, CHUNK), :])


MB = 512


def _mlp_body(e1_ref, e2_ref, et_ref, w1a_ref, w1b_ref, w1t_ref,
              b1_ref, w2_ref, b2_ref, out_ref):
    h = jnp.dot(e1_ref[...], w1a_ref[...], preferred_element_type=jnp.float32)
    h = h + jnp.dot(e2_ref[...], w1b_ref[...], preferred_element_type=jnp.float32)
    h = h + jnp.dot(et_ref[...], w1t_ref[...], preferred_element_type=jnp.float32)
    h = jnp.maximum(h + b1_ref[...], 0.0)
    out_ref[...] = jnp.dot(h, w2_ref[...], preferred_element_type=jnp.float32) + b2_ref[...]


_mlp = pl.pallas_call(
    _mlp_body,
    grid=(B // MB,),
    in_specs=[
        pl.BlockSpec((MB, MAIN), lambda i: (i, 0)),
        pl.BlockSpec((MB, MAIN), lambda i: (i, 0)),
        pl.BlockSpec((MB, TP), lambda i: (i, 0)),
        pl.BlockSpec((MAIN, 2 * D), lambda i: (0, 0)),
        pl.BlockSpec((MAIN, 2 * D), lambda i: (0, 0)),
        pl.BlockSpec((TP, 2 * D), lambda i: (0, 0)),
        pl.BlockSpec((1, 2 * D), lambda i: (0, 0)),
        pl.BlockSpec((2 * D, 1), lambda i: (0, 0)),
        pl.BlockSpec((1, 1), lambda i: (0, 0)),
    ],
    out_specs=pl.BlockSpec((MB, 1), lambda i: (i, 0)),
    out_shape=jax.ShapeDtypeStruct((B, 1), jnp.float32),
)


def kernel(input, lut, static_lut, W1, b1, W2, b2):
    tail = jnp.pad(
        jnp.concatenate([lut[:, MAIN:], static_lut[:, MAIN:]], axis=1),
        ((0, 0), (0, TP - 2 * TAIL)))
    inpr = (input.reshape(NJ, 2, NW, NCHUNK, CHUNK)
            .transpose(2, 3, 0, 1, 4)
            .reshape(NW, NCHUNK, NJ, GR))
    e1m, e2m, et = _sc_pool(inpr, lut, static_lut, tail)
    w1t = jnp.pad(
        jnp.concatenate([W1[MAIN:D], W1[D + MAIN:]], axis=0),
        ((0, TP - 2 * TAIL), (0, 0)))
    out = _mlp(e1m, e2m, et, W1[:MAIN], W1[D:D + MAIN], w1t,
               b1.reshape(1, 2 * D), W2, b2.reshape(1, 1))
    return out.reshape(B)


# speedup vs baseline: 1.4546x; 1.0306x over previous
import functools

import jax
import jax.numpy as jnp
from jax import lax
from jax.experimental import pallas as pl
from jax.experimental.pallas import tpu as pltpu
from jax.experimental.pallas import tpu_sc as plsc

V = 100000
D = 300
SEQ = 50
B = 4096

MAIN = 256
TAIL = D - MAIN
TP = 128

NC = 2
NS = 16
NW = NC * NS
BPW = B // NW
CHUNK = 64
NCHUNK = BPW // CHUNK
NJ = SEQ // 2
GR = 2 * CHUNK

LANES = 16
UNROLL = 1

_sc_mesh = plsc.VectorSubcoreMesh(
    core_axis_name="c", subcore_axis_name="s", num_cores=NC, num_subcores=NS
)


@functools.partial(
    pl.kernel,
    out_type=(
        jax.ShapeDtypeStruct((B, MAIN), jnp.float32),
        jax.ShapeDtypeStruct((B, MAIN), jnp.float32),
        jax.ShapeDtypeStruct((B, TP), jnp.float32),
    ),
    mesh=_sc_mesh,
    scratch_types=[
        pltpu.VMEM((NJ, GR), jnp.int32),
        [pltpu.VMEM((GR, MAIN), jnp.float32) for _ in range(2)],
        [pltpu.VMEM((GR, TP), jnp.float32) for _ in range(2)],
        pltpu.VMEM((CHUNK, MAIN), jnp.float32),
        pltpu.VMEM((CHUNK, TP), jnp.float32),
        [pltpu.SemaphoreType.DMA for _ in range(4)],
    ],
)
def _sc_pool(inpr_hbm, lut_hbm, slut_hbm, tail_hbm,
             e1_hbm, e2_hbm, et_hbm,
             idx_c, bufM, bufT, accM, accT, sems):
    wid = lax.axis_index("s") * NC + lax.axis_index("c")
    zero = jnp.zeros((LANES,), jnp.float32)

    def issue_main(tbl, j, slot):
        pltpu.async_copy(tbl.at[:, pl.ds(0, MAIN)].at[idx_c.at[j]],
                         bufM[slot], sems[slot])

    def wait_main(slot):
        pltpu.make_async_copy(
            lut_hbm.at[:, pl.ds(0, MAIN)].at[idx_c.at[0]],
            bufM[slot], sems[slot]).wait()

    def issue_tail(j, slot):
        pltpu.async_copy(tail_hbm.at[idx_c.at[j]], bufT[slot], sems[2 + slot])

    def wait_tail(slot):
        pltpu.make_async_copy(tail_hbm.at[idx_c.at[0]], bufT[slot],
                              sems[2 + slot]).wait()

    def accum_main(slot):
        def r_body(r, inner):
            for k in range(MAIN // LANES):
                o = pl.ds(k * LANES, LANES)
                x = bufM[slot][r, o] + bufM[slot][CHUNK + r, o]
                plsc.addupdate(accM.at[r, o], x)
            return inner
        lax.fori_loop(0, CHUNK, r_body, 0, unroll=UNROLL)

    def accum_tail(slot):
        def r_body(r, inner):
            for k in range(TP // LANES):
                o = pl.ds(k * LANES, LANES)
                x = bufT[slot][r, o] + bufT[slot][CHUNK + r, o]
                plsc.addupdate(accT.at[r, o], x)
            return inner
        lax.fori_loop(0, CHUNK, r_body, 0, unroll=UNROLL)

    def run_pass(tbl, with_tail):
        issue_main(tbl, 0, 0)
        if with_tail:
            issue_tail(0, 0)

        def zero_body(r, carry):
            for k in range(MAIN // LANES):
                accM[r, pl.ds(k * LANES, LANES)] = zero
            if with_tail:
                for k in range(TP // LANES):
                    accT[r, pl.ds(k * LANES, LANES)] = zero
            return carry

        lax.fori_loop(0, CHUNK, zero_body, 0, unroll=UNROLL)

        def j_body(j, carry):
            for par in range(2):
                jj = 2 * j + par

                @pl.when(jj + 1 < NJ)
                def _():
                    issue_main(tbl, jj + 1, (par + 1) % 2)
                    if with_tail:
                        issue_tail(jj + 1, (par + 1) % 2)

                wait_main(par)
                accum_main(par)
                if with_tail:
                    wait_tail(par)
                    accum_tail(par)
            return carry

        lax.fori_loop(0, NJ // 2, j_body, 0)
        wait_main(0)
        accum_main(0)
        if with_tail:
            wait_tail(0)
            accum_tail(0)

    for c in range(NCHUNK):
        base = wid * BPW + c * CHUNK
        pltpu.sync_copy(inpr_hbm.at[wid, c], idx_c)
        run_pass(lut_hbm, True)
        pltpu.sync_copy(accM, e1_hbm.at[pl.ds(base, CHUNK), :])
        pltpu.sync_copy(accT, et_hbm.at[pl.ds(base, CHUNK), :])
        run_pass(slut_hbm, False)
        pltpu.sync_copy(accM, e2_hbm.at[pl.ds(base, CHUNK), :])


MB = 512


def _mlp_body(e1_ref, e2_ref, et_ref, w1a_ref, w1b_ref, w1t_ref,
              b1_ref, w2_ref, b2_ref, out_ref):
    h = jnp.dot(e1_ref[...], w1a_ref[...], preferred_element_type=jnp.float32)
    h = h + jnp.dot(e2_ref[...], w1b_ref[...], preferred_element_type=jnp.float32)
    h = h + jnp.dot(et_ref[...], w1t_ref[...], preferred_element_type=jnp.float32)
    h = jnp.maximum(h + b1_ref[...], 0.0)
    out_ref[...] = jnp.dot(h, w2_ref[...], preferred_element_type=jnp.float32) + b2_ref[...]


_mlp = pl.pallas_call(
    _mlp_body,
    grid=(B // MB,),
    in_specs=[
        pl.BlockSpec((MB, MAIN), lambda i: (i, 0)),
        pl.BlockSpec((MB, MAIN), lambda i: (i, 0)),
        pl.BlockSpec((MB, TP), lambda i: (i, 0)),
        pl.BlockSpec((MAIN, 2 * D), lambda i: (0, 0)),
        pl.BlockSpec((MAIN, 2 * D), lambda i: (0, 0)),
        pl.BlockSpec((TP, 2 * D), lambda i: (0, 0)),
        pl.BlockSpec((1, 2 * D), lambda i: (0, 0)),
        pl.BlockSpec((2 * D, 1), lambda i: (0, 0)),
        pl.BlockSpec((1, 1), lambda i: (0, 0)),
    ],
    out_specs=pl.BlockSpec((MB, 1), lambda i: (i, 0)),
    out_shape=jax.ShapeDtypeStruct((B, 1), jnp.float32),
)


def kernel(input, lut, static_lut, W1, b1, W2, b2):
    tail = jnp.pad(
        jnp.concatenate([lut[:, MAIN:], static_lut[:, MAIN:]], axis=1),
        ((0, 0), (0, TP - 2 * TAIL)))
    inpr = (input.reshape(NJ, 2, NW, NCHUNK, CHUNK)
            .transpose(2, 3, 0, 1, 4)
            .reshape(NW, NCHUNK, NJ, GR))
    e1m, e2m, et = _sc_pool(inpr, lut, static_lut, tail)
    w1t = jnp.pad(
        jnp.concatenate([W1[MAIN:D], W1[D + MAIN:]], axis=0),
        ((0, TP - 2 * TAIL), (0, 0)))
    out = _mlp(e1m, e2m, et, W1[:MAIN], W1[D:D + MAIN], w1t,
               b1.reshape(1, 2 * D), W2, b2.reshape(1, 1))
    return out.reshape(B)
